# baseline (speedup 1.0000x reference)
import functools

import jax
import jax.numpy as jnp
from jax import lax
from jax.experimental import pallas as pl
from jax.experimental.pallas import tpu as pltpu
from jax.experimental.pallas import tpu_sc as plsc

NC = 2
NS = 16
CH = 80


def _ln_body(x_ref, g_ref, b_ref, o_ref):
    x = x_ref[...]
    mu = jnp.mean(x, axis=-1, keepdims=True)
    xc = x - mu
    var = jnp.mean(xc * xc, axis=-1, keepdims=True)
    o_ref[...] = xc * lax.rsqrt(var + 1e-5) * g_ref[...] + b_ref[...]


def _mlp_body(e_ref, c_ref, n_ref, w1_ref, b1_ref, w2_ref, b2_ref, msg_ref):
    n = n_ref[...]
    s = e_ref[...] + c_ref[...] + n
    s = s * jax.nn.sigmoid(s)
    h = jnp.dot(s.astype(jnp.bfloat16), w1_ref[...],
                preferred_element_type=jnp.float32) + b1_ref[...]
    h = h * jax.nn.sigmoid(h)
    t = jnp.dot(h.astype(jnp.bfloat16), w2_ref[...],
                preferred_element_type=jnp.float32) + b2_ref[...]
    msg_ref[...] = n * t


def _out_body(x_ref, *rest):
    aggs = rest[:-3]
    wt_ref, bt_ref, o_ref = rest[-3:]
    t = x_ref[...]
    for a_ref in aggs:
        t = t + a_ref[0] + a_ref[1]
    t = t * jax.nn.sigmoid(t)
    o_ref[...] = jnp.dot(t, wt_ref[...], preferred_element_type=jnp.float32) + bt_ref[...]


def _sc_mesh():
    return plsc.VectorSubcoreMesh(
        core_axis_name="c", subcore_axis_name="s", num_cores=NC, num_subcores=NS)


NBUF = 4


@functools.lru_cache(maxsize=None)
def _make_gather(N, D, E, e_off=0):
    NW = NC * NS
    EW = E // NW
    n_chunks = EW // CH
    n_outer = (n_chunks + NBUF - 1) // NBUF
    assert n_chunks >= NBUF

    DP = D // 2
    scratch = (
        [pltpu.VMEM((EW,), jnp.int32)] * 2
        + [pltpu.VMEM((CH, DP), jnp.int32)] * (2 * NBUF)
        + [pltpu.SemaphoreType.DMA] * (4 * NBUF)
    )

    @functools.partial(
        pl.kernel,
        out_type=(jax.ShapeDtypeStruct((E, DP), jnp.int32),
                  jax.ShapeDtypeStruct((E, DP), jnp.int32)),
        mesh=_sc_mesh(),
        compiler_params=pltpu.CompilerParams(use_tc_tiling_on_sc=False),
        scratch_types=scratch)
    def gather_k(x_hbm, cidx_hbm, nidx_hbm, cout_hbm, nout_hbm, *scr):
        cidx_all, nidx_all = scr[0], scr[1]
        crow = scr[2:2 + NBUF]
        nrow = scr[2 + NBUF:2 + 2 * NBUF]
        scro = scr[2 + 2 * NBUF:]
        gc = scro[0:NBUF]
        gn = scro[NBUF:2 * NBUF]
        wc = scro[2 * NBUF:3 * NBUF]
        wn = scro[3 * NBUF:4 * NBUF]
        wid = lax.axis_index("s") * NC + lax.axis_index("c")
        e0 = wid * EW
        pltpu.sync_copy(cidx_hbm.at[pl.ds(e_off + e0, EW)], cidx_all)
        pltpu.sync_copy(nidx_hbm.at[pl.ds(e_off + e0, EW)], nidx_all)

        def g_start(i, b):
            pltpu.async_copy(
                x_hbm.at[cidx_all.at[pl.ds(i * CH, CH)]], crow[b], gc[b])
            pltpu.async_copy(
                x_hbm.at[nidx_all.at[pl.ds(i * CH, CH)]], nrow[b], gn[b])

        def g_wait(b):
            pltpu.make_async_copy(x_hbm.at[pl.ds(0, CH), :], crow[b], gc[b]).wait()
            pltpu.make_async_copy(x_hbm.at[pl.ds(0, CH), :], nrow[b], gn[b]).wait()

        def w_start(i, b):
            pltpu.async_copy(crow[b], cout_hbm.at[pl.ds(e0 + i * CH, CH), :], wc[b])
            pltpu.async_copy(nrow[b], nout_hbm.at[pl.ds(e0 + i * CH, CH), :], wn[b])

        def w_wait(b):
            pltpu.make_async_copy(crow[b], cout_hbm.at[pl.ds(0, CH), :], wc[b]).wait()
            pltpu.make_async_copy(nrow[b], nout_hbm.at[pl.ds(0, CH), :], wn[b]).wait()

        for b in range(NBUF):
            g_start(b, b)

        def outer(j, carry):
            for b in range(NBUF):
                i = j * NBUF + b

                @pl.when(i < n_chunks)
                def _():
                    g_wait(b)
                    w_start(i, b)
                    w_wait(b)

                    @pl.when(i + NBUF < n_chunks)
                    def _():
                        g_start(i + NBUF, b)
            return carry

        lax.fori_loop(0, n_outer, outer, 0)

    return gather_k


@functools.lru_cache(maxsize=None)
def _make_scatter(N, D, EP, e_off=0):
    NW = NC * NS
    P = 1
    EW = EP // NW
    n_chunks = EW // CH
    NR = ((N // NS + 7) // 8) * 8
    NR_LAST = N - NR * (NS - 1)
    assert NR_LAST > 0 and NR_LAST % 8 == 0

    n_outer = (n_chunks + NBUF - 1) // NBUF
    assert n_chunks >= NBUF
    scratch = (
        [pltpu.VMEM((CH,), jnp.int32)] * NBUF
        + [pltpu.VMEM((CH, D), jnp.float32)] * NBUF
        + [pltpu.VMEM_SHARED((N, D), jnp.float32)]
        + [pltpu.SemaphoreType.DMA] * (3 * NBUF)
    )

    @functools.partial(
        pl.kernel,
        out_type=jax.ShapeDtypeStruct((NC, N, D), jnp.float32),
        mesh=_sc_mesh(),
        scratch_types=scratch)
    def scatter_k(cidx_hbm, zeros_hbm, *rest):
        msgs_hbm = rest[:P]
        agg_hbm = rest[P]
        scr = rest[P + 1:]
        cidx_v = scr[0:NBUF]
        msg_v = scr[NBUF:2 * NBUF]
        agg_sh = scr[2 * NBUF]
        ic = scr[2 * NBUF + 1:2 * NBUF + 1 + NBUF]
        im = scr[2 * NBUF + 1 + NBUF:2 * NBUF + 1 + 2 * NBUF]
        ss = scr[2 * NBUF + 1 + 2 * NBUF:2 * NBUF + 1 + 3 * NBUF]
        c = lax.axis_index("c")
        s = lax.axis_index("s")
        wid = s * NC + c
        e0 = wid * EW
        @pl.when(s < NS - 1)
        def _():
            pltpu.sync_copy(zeros_hbm.at[pl.ds(s * NR, NR), :],
                            agg_sh.at[pl.ds(s * NR, NR), :])

        @pl.when(s == NS - 1)
        def _():
            pltpu.sync_copy(zeros_hbm.at[pl.ds((NS - 1) * NR, NR_LAST), :],
                            agg_sh.at[pl.ds((NS - 1) * NR, NR_LAST), :])

        plsc.subcore_barrier()

        def l_wait(b):
            pltpu.make_async_copy(cidx_hbm.at[pl.ds(0, CH)], cidx_v[b], ic[b]).wait()
            pltpu.make_async_copy(
                msgs_hbm[0].at[pl.ds(0, CH), :], msg_v[b], im[b]).wait()

        def s_start(b):
            pltpu.async_copy(msg_v[b], agg_sh.at[cidx_v[b]], ss[b], add=True)

        def s_wait(b):
            pltpu.make_async_copy(
                msgs_hbm[0].at[pl.ds(0, CH), :], msg_v[b], ss[b]).wait()

        for p in range(P):
            msg_hbm = msgs_hbm[p]

            def l_start(i, b, p=p, msg_hbm=msg_hbm):
                pltpu.async_copy(
                    cidx_hbm.at[pl.ds(e_off + e0 + i * CH, CH)], cidx_v[b], ic[b])
                pltpu.async_copy(
                    msg_hbm.at[pl.ds(e0 + i * CH, CH), :], msg_v[b], im[b])

            for b in range(NBUF):
                l_start(b, b)

            def outer(j, carry, l_start=l_start):
                for b in range(NBUF):
                    i = j * NBUF + b

                    @pl.when(i < n_chunks)
                    def _():
                        l_wait(b)
                        s_start(b)
                        s_wait(b)

                        @pl.when(i + NBUF < n_chunks)
                        def _():
                            l_start(i + NBUF, b)
                return carry

            lax.fori_loop(0, n_outer, outer, 0)
        plsc.subcore_barrier()

        @pl.when(s < NS - 1)
        def _():
            pltpu.sync_copy(agg_sh.at[pl.ds(s * NR, NR), :],
                            agg_hbm.at[c, pl.ds(s * NR, NR), :])

        @pl.when(s == NS - 1)
        def _():
            pltpu.sync_copy(agg_sh.at[pl.ds((NS - 1) * NR, NR_LAST), :],
                            agg_hbm.at[c, pl.ds((NS - 1) * NR, NR_LAST), :])

    return scatter_k


def kernel(node_embeddings, edge_embeddings, edge_index_list, ln_g, ln_b,
           phi_w1, phi_b1, phi_w2, phi_b2, theta_w, theta_b):
    N, D = node_embeddings.shape
    E = edge_index_list.shape[1]
    H = phi_w1.shape[0]
    assert E % (NC * NS * CH) == 0 and N % NS == 0

    BN = 1000
    assert N % BN == 0
    x = pl.pallas_call(
        _ln_body,
        grid=(N // BN,),
        in_specs=[
            pl.BlockSpec((BN, D), lambda i: (i, 0)),
            pl.BlockSpec((1, D), lambda i: (0, 0)),
            pl.BlockSpec((1, D), lambda i: (0, 0)),
        ],
        out_specs=pl.BlockSpec((BN, D), lambda i: (i, 0)),
        out_shape=jax.ShapeDtypeStruct((N, D), jnp.float32),
    )(node_embeddings, ln_g.reshape(1, D), ln_b.reshape(1, D))

    P = 5
    EP = E // P
    cidx = edge_index_list[0]
    nidx = edge_index_list[1]
    BE = 1280
    assert EP % BE == 0 and E % (P * NC * NS * CH) == 0
    w1t = phi_w1.T.astype(jnp.bfloat16)
    w2t = phi_w2.T.astype(jnp.bfloat16)

    def mlp_fn(p):
        off = p * (EP // BE)
        return pl.pallas_call(
            _mlp_body,
            grid=(EP // BE,),
            in_specs=[
                pl.BlockSpec((BE, D), lambda i: (off + i, 0)),
                pl.BlockSpec((BE, D), lambda i: (i, 0)),
                pl.BlockSpec((BE, D), lambda i: (i, 0)),
                pl.BlockSpec((D, H), lambda i: (0, 0)),
                pl.BlockSpec((1, H), lambda i: (0, 0)),
                pl.BlockSpec((H, D), lambda i: (0, 0)),
                pl.BlockSpec((1, D), lambda i: (0, 0)),
            ],
            out_specs=pl.BlockSpec((BE, D), lambda i: (i, 0)),
            out_shape=jax.ShapeDtypeStruct((EP, D), jnp.float32),
            compiler_params=pltpu.CompilerParams(
                dimension_semantics=("arbitrary",)),
        )

    zeros = jnp.zeros((N, D), jnp.float32)
    msgs = []
    aggps = []
    tok = x[0, 0]

    def launch_scatter(q, tok):
        cidx_q, _ = lax.optimization_barrier((cidx, tok))
        aggps.append(_make_scatter(N, D, EP, q * EP)(cidx_q, zeros, msgs[q]))
        return aggps[-1][0, 0, 0]

    xp = lax.bitcast_convert_type(
        x.astype(jnp.bfloat16).reshape(N, D // 2, 2), jnp.int32)
    for p in range(P):
        cidx_p, _ = lax.optimization_barrier((cidx, tok))
        cpack, npack = _make_gather(N, D, EP, p * EP)(xp, cidx_p, nidx)
        tok = cpack[0, 0]
        crows = lax.bitcast_convert_type(cpack, jnp.bfloat16).reshape(EP, D)
        nrows = lax.bitcast_convert_type(npack, jnp.bfloat16).reshape(EP, D)
        msgs.append(mlp_fn(p)(
            edge_embeddings, crows, nrows, w1t, phi_b1.reshape(1, H),
            w2t, phi_b2.reshape(1, D)))
        if p >= 1:
            tok = launch_scatter(p - 1, tok)
    tok = launch_scatter(P - 1, tok)

    wt = theta_w.T
    out = pl.pallas_call(
        _out_body,
        grid=(N // BN,),
        in_specs=(
            [pl.BlockSpec((BN, D), lambda i: (i, 0))]
            + [pl.BlockSpec((NC, BN, D), lambda i: (0, i, 0))] * P
            + [pl.BlockSpec((D, D), lambda i: (0, 0)),
               pl.BlockSpec((1, D), lambda i: (0, 0))]
        ),
        out_specs=pl.BlockSpec((BN, D), lambda i: (i, 0)),
        out_shape=jax.ShapeDtypeStruct((N, D), jnp.float32),
    )(x, *aggps, wt, theta_b.reshape(1, D))
    return out


# speedup vs baseline: 2.8149x; 2.8149x over previous
import functools

import jax
import jax.numpy as jnp
from jax import lax
from jax.experimental import pallas as pl
from jax.experimental.pallas import tpu as pltpu
from jax.experimental.pallas import tpu_sc as plsc

NC = 2
NS = 16
CH = 80


def _ln_body(x_ref, g_ref, b_ref, o_ref):
    x = x_ref[...]
    mu = jnp.mean(x, axis=-1, keepdims=True)
    xc = x - mu
    var = jnp.mean(xc * xc, axis=-1, keepdims=True)
    o_ref[...] = xc * lax.rsqrt(var + 1e-5) * g_ref[...] + b_ref[...]


def _unpack_rows(u):
    lo = lax.bitcast_convert_type(jnp.left_shift(u, 16), jnp.float32)
    hi = lax.bitcast_convert_type(
        jnp.bitwise_and(u, jnp.int32(-65536)), jnp.float32)
    return jnp.concatenate([lo, hi], axis=1)


def _mlp_body(e_ref, c_ref, n_ref, w1_ref, b1_ref, w2_ref, b2_ref, msg_ref):
    n = _unpack_rows(n_ref[...])
    s = e_ref[...] + _unpack_rows(c_ref[...]) + n
    s = s * jax.nn.sigmoid(s)
    h = jnp.dot(s.astype(jnp.bfloat16), w1_ref[...],
                preferred_element_type=jnp.float32) + b1_ref[...]
    h = h * jax.nn.sigmoid(h)
    t = jnp.dot(h.astype(jnp.bfloat16), w2_ref[...],
                preferred_element_type=jnp.float32) + b2_ref[...]
    msg_ref[...] = n * t


def _out_body(x_ref, *rest):
    aggs = rest[:-3]
    wt_ref, bt_ref, o_ref = rest[-3:]
    t = x_ref[...]
    for a_ref in aggs:
        t = t + a_ref[0] + a_ref[1]
    t = t * jax.nn.sigmoid(t)
    o_ref[...] = jnp.dot(t, wt_ref[...], preferred_element_type=jnp.float32) + bt_ref[...]


def _sc_mesh():
    return plsc.VectorSubcoreMesh(
        core_axis_name="c", subcore_axis_name="s", num_cores=NC, num_subcores=NS)


NBUF = 4


@functools.lru_cache(maxsize=None)
def _make_gather(N, D, E, e_off=0):
    NW = NC * NS
    EW = E // NW
    n_chunks = EW // CH
    n_outer = (n_chunks + NBUF - 1) // NBUF
    assert n_chunks >= NBUF

    DP = D // 2
    scratch = (
        [pltpu.VMEM((EW,), jnp.int32)] * 2
        + [pltpu.VMEM((CH, DP), jnp.int32)] * (2 * NBUF)
        + [pltpu.SemaphoreType.DMA] * (4 * NBUF)
    )

    @functools.partial(
        pl.kernel,
        out_type=(jax.ShapeDtypeStruct((E, DP), jnp.int32),
                  jax.ShapeDtypeStruct((E, DP), jnp.int32)),
        mesh=_sc_mesh(),
        compiler_params=pltpu.CompilerParams(use_tc_tiling_on_sc=False),
        scratch_types=scratch)
    def gather_k(x_hbm, cidx_hbm, nidx_hbm, cout_hbm, nout_hbm, *scr):
        cidx_all, nidx_all = scr[0], scr[1]
        crow = scr[2:2 + NBUF]
        nrow = scr[2 + NBUF:2 + 2 * NBUF]
        scro = scr[2 + 2 * NBUF:]
        gc = scro[0:NBUF]
        gn = scro[NBUF:2 * NBUF]
        wc = scro[2 * NBUF:3 * NBUF]
        wn = scro[3 * NBUF:4 * NBUF]
        wid = lax.axis_index("s") * NC + lax.axis_index("c")
        e0 = wid * EW
        pltpu.sync_copy(cidx_hbm.at[pl.ds(e_off + e0, EW)], cidx_all)
        pltpu.sync_copy(nidx_hbm.at[pl.ds(e_off + e0, EW)], nidx_all)

        def g_start(i, b):
            pltpu.async_copy(
                x_hbm.at[cidx_all.at[pl.ds(i * CH, CH)]], crow[b], gc[b])
            pltpu.async_copy(
                x_hbm.at[nidx_all.at[pl.ds(i * CH, CH)]], nrow[b], gn[b])

        def g_wait(b):
            pltpu.make_async_copy(x_hbm.at[pl.ds(0, CH), :], crow[b], gc[b]).wait()
            pltpu.make_async_copy(x_hbm.at[pl.ds(0, CH), :], nrow[b], gn[b]).wait()

        def w_start(i, b):
            pltpu.async_copy(crow[b], cout_hbm.at[pl.ds(e0 + i * CH, CH), :], wc[b])
            pltpu.async_copy(nrow[b], nout_hbm.at[pl.ds(e0 + i * CH, CH), :], wn[b])

        def w_wait(b):
            pltpu.make_async_copy(crow[b], cout_hbm.at[pl.ds(0, CH), :], wc[b]).wait()
            pltpu.make_async_copy(nrow[b], nout_hbm.at[pl.ds(0, CH), :], wn[b]).wait()

        for b in range(NBUF):
            g_start(b, b)

        def outer(j, carry):
            for b in range(NBUF):
                i = j * NBUF + b

                @pl.when(i < n_chunks)
                def _():
                    g_wait(b)
                    w_start(i, b)
                    w_wait(b)

                    @pl.when(i + NBUF < n_chunks)
                    def _():
                        g_start(i + NBUF, b)
            return carry

        lax.fori_loop(0, n_outer, outer, 0)

    return gather_k


@functools.lru_cache(maxsize=None)
def _make_scatter(N, D, EP, e_off=0):
    NW = NC * NS
    P = 1
    EW = EP // NW
    n_chunks = EW // CH
    NR = ((N // NS + 7) // 8) * 8
    NR_LAST = N - NR * (NS - 1)
    assert NR_LAST > 0 and NR_LAST % 8 == 0

    n_outer = (n_chunks + NBUF - 1) // NBUF
    assert n_chunks >= NBUF
    scratch = (
        [pltpu.VMEM((CH,), jnp.int32)] * NBUF
        + [pltpu.VMEM((CH, D), jnp.float32)] * NBUF
        + [pltpu.VMEM_SHARED((N, D), jnp.float32)]
        + [pltpu.SemaphoreType.DMA] * (3 * NBUF)
    )

    @functools.partial(
        pl.kernel,
        out_type=jax.ShapeDtypeStruct((NC, N, D), jnp.float32),
        mesh=_sc_mesh(),
        scratch_types=scratch)
    def scatter_k(cidx_hbm, zeros_hbm, *rest):
        msgs_hbm = rest[:P]
        agg_hbm = rest[P]
        scr = rest[P + 1:]
        cidx_v = scr[0:NBUF]
        msg_v = scr[NBUF:2 * NBUF]
        agg_sh = scr[2 * NBUF]
        ic = scr[2 * NBUF + 1:2 * NBUF + 1 + NBUF]
        im = scr[2 * NBUF + 1 + NBUF:2 * NBUF + 1 + 2 * NBUF]
        ss = scr[2 * NBUF + 1 + 2 * NBUF:2 * NBUF + 1 + 3 * NBUF]
        c = lax.axis_index("c")
        s = lax.axis_index("s")
        wid = s * NC + c
        e0 = wid * EW
        @pl.when(s < NS - 1)
        def _():
            pltpu.sync_copy(zeros_hbm.at[pl.ds(s * NR, NR), :],
                            agg_sh.at[pl.ds(s * NR, NR), :])

        @pl.when(s == NS - 1)
        def _():
            pltpu.sync_copy(zeros_hbm.at[pl.ds((NS - 1) * NR, NR_LAST), :],
                            agg_sh.at[pl.ds((NS - 1) * NR, NR_LAST), :])

        plsc.subcore_barrier()

        def l_wait(b):
            pltpu.make_async_copy(cidx_hbm.at[pl.ds(0, CH)], cidx_v[b], ic[b]).wait()
            pltpu.make_async_copy(
                msgs_hbm[0].at[pl.ds(0, CH), :], msg_v[b], im[b]).wait()

        def s_start(b):
            pltpu.async_copy(msg_v[b], agg_sh.at[cidx_v[b]], ss[b], add=True)

        def s_wait(b):
            pltpu.make_async_copy(
                msgs_hbm[0].at[pl.ds(0, CH), :], msg_v[b], ss[b]).wait()

        for p in range(P):
            msg_hbm = msgs_hbm[p]

            def l_start(i, b, p=p, msg_hbm=msg_hbm):
                pltpu.async_copy(
                    cidx_hbm.at[pl.ds(e_off + e0 + i * CH, CH)], cidx_v[b], ic[b])
                pltpu.async_copy(
                    msg_hbm.at[pl.ds(e0 + i * CH, CH), :], msg_v[b], im[b])

            for b in range(NBUF):
                l_start(b, b)

            def outer(j, carry, l_start=l_start):
                for b in range(NBUF):
                    i = j * NBUF + b

                    @pl.when(i < n_chunks)
                    def _():
                        l_wait(b)
                        s_start(b)
                        s_wait(b)

                        @pl.when(i + NBUF < n_chunks)
                        def _():
                            l_start(i + NBUF, b)
                return carry

            lax.fori_loop(0, n_outer, outer, 0)
        plsc.subcore_barrier()

        @pl.when(s < NS - 1)
        def _():
            pltpu.sync_copy(agg_sh.at[pl.ds(s * NR, NR), :],
                            agg_hbm.at[c, pl.ds(s * NR, NR), :])

        @pl.when(s == NS - 1)
        def _():
            pltpu.sync_copy(agg_sh.at[pl.ds((NS - 1) * NR, NR_LAST), :],
                            agg_hbm.at[c, pl.ds((NS - 1) * NR, NR_LAST), :])

    return scatter_k


def kernel(node_embeddings, edge_embeddings, edge_index_list, ln_g, ln_b,
           phi_w1, phi_b1, phi_w2, phi_b2, theta_w, theta_b):
    N, D = node_embeddings.shape
    E = edge_index_list.shape[1]
    H = phi_w1.shape[0]
    assert E % (NC * NS * CH) == 0 and N % NS == 0

    BN = 1000
    assert N % BN == 0
    x = pl.pallas_call(
        _ln_body,
        grid=(N // BN,),
        in_specs=[
            pl.BlockSpec((BN, D), lambda i: (i, 0)),
            pl.BlockSpec((1, D), lambda i: (0, 0)),
            pl.BlockSpec((1, D), lambda i: (0, 0)),
        ],
        out_specs=pl.BlockSpec((BN, D), lambda i: (i, 0)),
        out_shape=jax.ShapeDtypeStruct((N, D), jnp.float32),
    )(node_embeddings, ln_g.reshape(1, D), ln_b.reshape(1, D))

    P = 5
    EP = E // P
    cidx = edge_index_list[0]
    nidx = edge_index_list[1]
    BE = 1280
    assert EP % BE == 0 and E % (P * NC * NS * CH) == 0
    w1t = phi_w1.T.astype(jnp.bfloat16)
    w2t = phi_w2.T.astype(jnp.bfloat16)

    def mlp_fn(p):
        off = p * (EP // BE)
        return pl.pallas_call(
            _mlp_body,
            grid=(EP // BE,),
            in_specs=[
                pl.BlockSpec((BE, D), lambda i: (off + i, 0)),
                pl.BlockSpec((BE, D // 2), lambda i: (i, 0)),
                pl.BlockSpec((BE, D // 2), lambda i: (i, 0)),
                pl.BlockSpec((D, H), lambda i: (0, 0)),
                pl.BlockSpec((1, H), lambda i: (0, 0)),
                pl.BlockSpec((H, D), lambda i: (0, 0)),
                pl.BlockSpec((1, D), lambda i: (0, 0)),
            ],
            out_specs=pl.BlockSpec((BE, D), lambda i: (i, 0)),
            out_shape=jax.ShapeDtypeStruct((EP, D), jnp.float32),
            compiler_params=pltpu.CompilerParams(
                dimension_semantics=("arbitrary",)),
        )

    zeros = jnp.zeros((N, D), jnp.float32)
    msgs = []
    aggps = []
    tok = x[0, 0]

    def launch_scatter(q, tok):
        cidx_q, _ = lax.optimization_barrier((cidx, tok))
        aggps.append(_make_scatter(N, D, EP, q * EP)(cidx_q, zeros, msgs[q]))
        return aggps[-1][0, 0, 0]

    xb = x.astype(jnp.bfloat16)
    lo_u = lax.bitcast_convert_type(xb[:, :D // 2], jnp.uint16).astype(jnp.uint32)
    hi_u = lax.bitcast_convert_type(xb[:, D // 2:], jnp.uint16).astype(jnp.uint32)
    xp = lax.bitcast_convert_type(lo_u | (hi_u << 16), jnp.int32)
    for p in range(P):
        cidx_p, _ = lax.optimization_barrier((cidx, tok))
        cpack, npack = _make_gather(N, D, EP, p * EP)(xp, cidx_p, nidx)
        tok = cpack[0, 0]
        msgs.append(mlp_fn(p)(
            edge_embeddings, cpack, npack, w1t, phi_b1.reshape(1, H),
            w2t, phi_b2.reshape(1, D)))
        if p >= 1:
            tok = launch_scatter(p - 1, tok)
    tok = launch_scatter(P - 1, tok)

    wt = theta_w.T
    out = pl.pallas_call(
        _out_body,
        grid=(N // BN,),
        in_specs=(
            [pl.BlockSpec((BN, D), lambda i: (i, 0))]
            + [pl.BlockSpec((NC, BN, D), lambda i: (0, i, 0))] * P
            + [pl.BlockSpec((D, D), lambda i: (0, 0)),
               pl.BlockSpec((1, D), lambda i: (0, 0))]
        ),
        out_specs=pl.BlockSpec((BN, D), lambda i: (i, 0)),
        out_shape=jax.ShapeDtypeStruct((N, D), jnp.float32),
    )(x, *aggps, wt, theta_b.reshape(1, D))
    return out


# speedup vs baseline: 3.7945x; 1.3480x over previous
import functools

import jax
import jax.numpy as jnp
from jax import lax
from jax.experimental import pallas as pl
from jax.experimental.pallas import tpu as pltpu
from jax.experimental.pallas import tpu_sc as plsc

NC = 2
NS = 16
CH = 80


def _ln_body(x_ref, g_ref, b_ref, o_ref):
    x = x_ref[...]
    mu = jnp.mean(x, axis=-1, keepdims=True)
    xc = x - mu
    var = jnp.mean(xc * xc, axis=-1, keepdims=True)
    o_ref[...] = xc * lax.rsqrt(var + 1e-5) * g_ref[...] + b_ref[...]


def _ln_rows(t, g, b):
    mu = jnp.mean(t, axis=-1, keepdims=True)
    tc = t - mu
    var = jnp.mean(tc * tc, axis=-1, keepdims=True)
    return tc * lax.rsqrt(var + 1e-5) * g + b


def _mlp_body(e_ref, c_ref, n_ref, g_ref, b_ref, w1_ref, b1_ref, w2_ref,
              b2_ref, msg_ref):
    g = g_ref[...]
    b = b_ref[...]
    n = _ln_rows(n_ref[...], g, b)
    s = e_ref[...] + _ln_rows(c_ref[...], g, b) + n
    s = s * jax.nn.sigmoid(s)
    h = jnp.dot(s.astype(jnp.bfloat16), w1_ref[...],
                preferred_element_type=jnp.float32) + b1_ref[...]
    h = h * jax.nn.sigmoid(h)
    t = jnp.dot(h.astype(jnp.bfloat16), w2_ref[...],
                preferred_element_type=jnp.float32) + b2_ref[...]
    msg_ref[...] = n * t


def _out_body(x_ref, *rest):
    aggs = rest[:-3]
    wt_ref, bt_ref, o_ref = rest[-3:]
    t = x_ref[...]
    for a_ref in aggs:
        t = t + a_ref[0] + a_ref[1]
    t = t * jax.nn.sigmoid(t)
    o_ref[...] = jnp.dot(t, wt_ref[...], preferred_element_type=jnp.float32) + bt_ref[...]


def _sc_mesh():
    return plsc.VectorSubcoreMesh(
        core_axis_name="c", subcore_axis_name="s", num_cores=NC, num_subcores=NS)


NBUF = 4


@functools.lru_cache(maxsize=None)
def _make_gather(N, D, E, e_off=0):
    NW = NC * NS
    EW = E // NW
    n_chunks = EW // CH
    n_outer = (n_chunks + NBUF - 1) // NBUF
    assert n_chunks >= NBUF

    scratch = (
        [pltpu.VMEM((EW,), jnp.int32)] * 2
        + [pltpu.VMEM((CH, D), jnp.float32)] * (2 * NBUF)
        + [pltpu.SemaphoreType.DMA] * (4 * NBUF)
    )

    @functools.partial(
        pl.kernel,
        out_type=(jax.ShapeDtypeStruct((E, D), jnp.float32),
                  jax.ShapeDtypeStruct((E, D), jnp.float32)),
        mesh=_sc_mesh(),
        scratch_types=scratch)
    def gather_k(x_hbm, cidx_hbm, nidx_hbm, cout_hbm, nout_hbm, *scr):
        cidx_all, nidx_all = scr[0], scr[1]
        crow = scr[2:2 + NBUF]
        nrow = scr[2 + NBUF:2 + 2 * NBUF]
        scro = scr[2 + 2 * NBUF:]
        gc = scro[0:NBUF]
        gn = scro[NBUF:2 * NBUF]
        wc = scro[2 * NBUF:3 * NBUF]
        wn = scro[3 * NBUF:4 * NBUF]
        wid = lax.axis_index("s") * NC + lax.axis_index("c")
        e0 = wid * EW
        pltpu.sync_copy(cidx_hbm.at[pl.ds(e_off + e0, EW)], cidx_all)
        pltpu.sync_copy(nidx_hbm.at[pl.ds(e_off + e0, EW)], nidx_all)

        def g_start(i, b):
            pltpu.async_copy(
                x_hbm.at[cidx_all.at[pl.ds(i * CH, CH)]], crow[b], gc[b])
            pltpu.async_copy(
                x_hbm.at[nidx_all.at[pl.ds(i * CH, CH)]], nrow[b], gn[b])

        def g_wait(b):
            pltpu.make_async_copy(x_hbm.at[pl.ds(0, CH), :], crow[b], gc[b]).wait()
            pltpu.make_async_copy(x_hbm.at[pl.ds(0, CH), :], nrow[b], gn[b]).wait()

        def w_start(i, b):
            pltpu.async_copy(crow[b], cout_hbm.at[pl.ds(e0 + i * CH, CH), :], wc[b])
            pltpu.async_copy(nrow[b], nout_hbm.at[pl.ds(e0 + i * CH, CH), :], wn[b])

        def w_wait(b):
            pltpu.make_async_copy(crow[b], cout_hbm.at[pl.ds(0, CH), :], wc[b]).wait()
            pltpu.make_async_copy(nrow[b], nout_hbm.at[pl.ds(0, CH), :], wn[b]).wait()

        for b in range(NBUF):
            g_start(b, b)

        def outer(j, carry):
            for b in range(NBUF):
                i = j * NBUF + b

                @pl.when(i < n_chunks)
                def _():
                    g_wait(b)
                    w_start(i, b)
                    w_wait(b)

                    @pl.when(i + NBUF < n_chunks)
                    def _():
                        g_start(i + NBUF, b)
            return carry

        lax.fori_loop(0, n_outer, outer, 0)

    return gather_k


@functools.lru_cache(maxsize=None)
def _make_scatter(N, D, EP, e_off=0):
    NW = NC * NS
    P = 1
    EW = EP // NW
    n_chunks = EW // CH
    NR = ((N // NS + 7) // 8) * 8
    NR_LAST = N - NR * (NS - 1)
    assert NR_LAST > 0 and NR_LAST % 8 == 0

    n_outer = (n_chunks + NBUF - 1) // NBUF
    assert n_chunks >= NBUF
    scratch = (
        [pltpu.VMEM((CH,), jnp.int32)] * NBUF
        + [pltpu.VMEM((CH, D), jnp.float32)] * NBUF
        + [pltpu.VMEM_SHARED((N, D), jnp.float32)]
        + [pltpu.SemaphoreType.DMA] * (3 * NBUF)
    )

    @functools.partial(
        pl.kernel,
        out_type=jax.ShapeDtypeStruct((NC, N, D), jnp.float32),
        mesh=_sc_mesh(),
        scratch_types=scratch)
    def scatter_k(cidx_hbm, zeros_hbm, *rest):
        msgs_hbm = rest[:P]
        agg_hbm = rest[P]
        scr = rest[P + 1:]
        cidx_v = scr[0:NBUF]
        msg_v = scr[NBUF:2 * NBUF]
        agg_sh = scr[2 * NBUF]
        ic = scr[2 * NBUF + 1:2 * NBUF + 1 + NBUF]
        im = scr[2 * NBUF + 1 + NBUF:2 * NBUF + 1 + 2 * NBUF]
        ss = scr[2 * NBUF + 1 + 2 * NBUF:2 * NBUF + 1 + 3 * NBUF]
        c = lax.axis_index("c")
        s = lax.axis_index("s")
        wid = s * NC + c
        e0 = wid * EW
        @pl.when(s < NS - 1)
        def _():
            pltpu.sync_copy(zeros_hbm.at[pl.ds(s * NR, NR), :],
                            agg_sh.at[pl.ds(s * NR, NR), :])

        @pl.when(s == NS - 1)
        def _():
            pltpu.sync_copy(zeros_hbm.at[pl.ds((NS - 1) * NR, NR_LAST), :],
                            agg_sh.at[pl.ds((NS - 1) * NR, NR_LAST), :])

        plsc.subcore_barrier()

        def l_wait(b):
            pltpu.make_async_copy(cidx_hbm.at[pl.ds(0, CH)], cidx_v[b], ic[b]).wait()
            pltpu.make_async_copy(
                msgs_hbm[0].at[pl.ds(0, CH), :], msg_v[b], im[b]).wait()

        def s_start(b):
            pltpu.async_copy(msg_v[b], agg_sh.at[cidx_v[b]], ss[b], add=True)

        def s_wait(b):
            pltpu.make_async_copy(
                msgs_hbm[0].at[pl.ds(0, CH), :], msg_v[b], ss[b]).wait()

        for p in range(P):
            msg_hbm = msgs_hbm[p]

            def l_start(i, b, p=p, msg_hbm=msg_hbm):
                pltpu.async_copy(
                    cidx_hbm.at[pl.ds(e_off + e0 + i * CH, CH)], cidx_v[b], ic[b])
                pltpu.async_copy(
                    msg_hbm.at[pl.ds(e0 + i * CH, CH), :], msg_v[b], im[b])

            for b in range(NBUF):
                l_start(b, b)

            def outer(j, carry, l_start=l_start):
                for b in range(NBUF):
                    i = j * NBUF + b

                    @pl.when(i < n_chunks)
                    def _():
                        l_wait(b)
                        s_start(b)
                        s_wait(b)

                        @pl.when(i + NBUF < n_chunks)
                        def _():
                            l_start(i + NBUF, b)
                return carry

            lax.fori_loop(0, n_outer, outer, 0)
        plsc.subcore_barrier()

        @pl.when(s < NS - 1)
        def _():
            pltpu.sync_copy(agg_sh.at[pl.ds(s * NR, NR), :],
                            agg_hbm.at[c, pl.ds(s * NR, NR), :])

        @pl.when(s == NS - 1)
        def _():
            pltpu.sync_copy(agg_sh.at[pl.ds((NS - 1) * NR, NR_LAST), :],
                            agg_hbm.at[c, pl.ds((NS - 1) * NR, NR_LAST), :])

    return scatter_k


def kernel(node_embeddings, edge_embeddings, edge_index_list, ln_g, ln_b,
           phi_w1, phi_b1, phi_w2, phi_b2, theta_w, theta_b):
    N, D = node_embeddings.shape
    E = edge_index_list.shape[1]
    H = phi_w1.shape[0]
    assert E % (NC * NS * CH) == 0 and N % NS == 0

    BN = 1000
    assert N % BN == 0
    x = pl.pallas_call(
        _ln_body,
        grid=(N // BN,),
        in_specs=[
            pl.BlockSpec((BN, D), lambda i: (i, 0)),
            pl.BlockSpec((1, D), lambda i: (0, 0)),
            pl.BlockSpec((1, D), lambda i: (0, 0)),
        ],
        out_specs=pl.BlockSpec((BN, D), lambda i: (i, 0)),
        out_shape=jax.ShapeDtypeStruct((N, D), jnp.float32),
    )(node_embeddings, ln_g.reshape(1, D), ln_b.reshape(1, D))

    P = 5
    EP = E // P
    cidx = edge_index_list[0]
    nidx = edge_index_list[1]
    BE = 2560
    assert EP % BE == 0 and E % (P * NC * NS * CH) == 0
    w1t = phi_w1.T.astype(jnp.bfloat16)
    w2t = phi_w2.T.astype(jnp.bfloat16)

    def mlp_fn(p):
        off = p * (EP // BE)
        return pl.pallas_call(
            _mlp_body,
            grid=(EP // BE,),
            in_specs=[
                pl.BlockSpec((BE, D), lambda i: (off + i, 0)),
                pl.BlockSpec((BE, D), lambda i: (i, 0)),
                pl.BlockSpec((BE, D), lambda i: (i, 0)),
                pl.BlockSpec((1, D), lambda i: (0, 0)),
                pl.BlockSpec((1, D), lambda i: (0, 0)),
                pl.BlockSpec((D, H), lambda i: (0, 0)),
                pl.BlockSpec((1, H), lambda i: (0, 0)),
                pl.BlockSpec((H, D), lambda i: (0, 0)),
                pl.BlockSpec((1, D), lambda i: (0, 0)),
            ],
            out_specs=pl.BlockSpec((BE, D), lambda i: (i, 0)),
            out_shape=jax.ShapeDtypeStruct((EP, D), jnp.float32),
            compiler_params=pltpu.CompilerParams(
                dimension_semantics=("arbitrary",)),
        )

    zeros = jnp.zeros((N, D), jnp.float32)
    msgs = []
    aggps = []
    tok = x[0, 0]

    def launch_scatter(q, tok):
        cidx_q, _ = lax.optimization_barrier((cidx, tok))
        aggps.append(_make_scatter(N, D, EP, q * EP)(cidx_q, zeros, msgs[q]))
        return aggps[-1][0, 0, 0]

    tok = node_embeddings[0, 0]
    for p in range(P):
        cidx_p, _ = lax.optimization_barrier((cidx, tok))
        crows, nrows = _make_gather(N, D, EP, p * EP)(
            node_embeddings, cidx_p, nidx)
        tok = crows[0, 0]
        msgs.append(mlp_fn(p)(
            edge_embeddings, crows, nrows, ln_g.reshape(1, D),
            ln_b.reshape(1, D), w1t, phi_b1.reshape(1, H),
            w2t, phi_b2.reshape(1, D)))
        if p >= 1:
            tok = launch_scatter(p - 1, tok)
    tok = launch_scatter(P - 1, tok)

    wt = theta_w.T
    out = pl.pallas_call(
        _out_body,
        grid=(N // BN,),
        in_specs=(
            [pl.BlockSpec((BN, D), lambda i: (i, 0))]
            + [pl.BlockSpec((NC, BN, D), lambda i: (0, i, 0))] * P
            + [pl.BlockSpec((D, D), lambda i: (0, 0)),
               pl.BlockSpec((1, D), lambda i: (0, 0))]
        ),
        out_specs=pl.BlockSpec((BN, D), lambda i: (i, 0)),
        out_shape=jax.ShapeDtypeStruct((N, D), jnp.float32),
    )(x, *aggps, wt, theta_b.reshape(1, D))
    return out


# speedup vs baseline: 3.8023x; 1.0021x over previous
import functools

import jax
import jax.numpy as jnp
from jax import lax
from jax.experimental import pallas as pl
from jax.experimental.pallas import tpu as pltpu
from jax.experimental.pallas import tpu_sc as plsc

NC = 2
NS = 16
CH = 80


def _ln_body(x_ref, g_ref, b_ref, o_ref):
    x = x_ref[...]
    mu = jnp.mean(x, axis=-1, keepdims=True)
    xc = x - mu
    var = jnp.mean(xc * xc, axis=-1, keepdims=True)
    o_ref[...] = xc * lax.rsqrt(var + 1e-5) * g_ref[...] + b_ref[...]


def _ln_rows(t, g, b):
    mu = jnp.mean(t, axis=-1, keepdims=True)
    tc = t - mu
    var = jnp.mean(tc * tc, axis=-1, keepdims=True)
    return tc * lax.rsqrt(var + 1e-5) * g + b


def _mlp_body(e_ref, c_ref, n_ref, g_ref, b_ref, w1_ref, b1_ref, w2_ref,
              b2_ref, msg_ref):
    g = g_ref[...]
    b = b_ref[...]
    n = _ln_rows(n_ref[...], g, b)
    s = e_ref[...] + _ln_rows(c_ref[...], g, b) + n
    s = s * jax.nn.sigmoid(s)
    h = jnp.dot(s.astype(jnp.bfloat16), w1_ref[...],
                preferred_element_type=jnp.float32) + b1_ref[...]
    h = h * jax.nn.sigmoid(h)
    t = jnp.dot(h.astype(jnp.bfloat16), w2_ref[...],
                preferred_element_type=jnp.float32) + b2_ref[...]
    msg_ref[...] = n * t


def _out_body(x_ref, *rest):
    aggs = rest[:-3]
    wt_ref, bt_ref, o_ref = rest[-3:]
    t = x_ref[...]
    for a_ref in aggs:
        t = t + a_ref[0] + a_ref[1]
    t = t * jax.nn.sigmoid(t)
    o_ref[...] = jnp.dot(t, wt_ref[...], preferred_element_type=jnp.float32) + bt_ref[...]


def _sc_mesh():
    return plsc.VectorSubcoreMesh(
        core_axis_name="c", subcore_axis_name="s", num_cores=NC, num_subcores=NS)


NBUF = 5


@functools.lru_cache(maxsize=None)
def _make_gather(N, D, E, e_off=0):
    NW = NC * NS
    EW = E // NW
    n_chunks = EW // CH
    n_outer = (n_chunks + NBUF - 1) // NBUF
    assert n_chunks >= NBUF

    scratch = (
        [pltpu.VMEM((EW,), jnp.int32)] * 2
        + [pltpu.VMEM((CH, D), jnp.float32)] * (2 * NBUF)
        + [pltpu.SemaphoreType.DMA] * (4 * NBUF)
    )

    @functools.partial(
        pl.kernel,
        out_type=(jax.ShapeDtypeStruct((E, D), jnp.float32),
                  jax.ShapeDtypeStruct((E, D), jnp.float32)),
        mesh=_sc_mesh(),
        scratch_types=scratch)
    def gather_k(x_hbm, cidx_hbm, nidx_hbm, cout_hbm, nout_hbm, *scr):
        cidx_all, nidx_all = scr[0], scr[1]
        crow = scr[2:2 + NBUF]
        nrow = scr[2 + NBUF:2 + 2 * NBUF]
        scro = scr[2 + 2 * NBUF:]
        gc = scro[0:NBUF]
        gn = scro[NBUF:2 * NBUF]
        wc = scro[2 * NBUF:3 * NBUF]
        wn = scro[3 * NBUF:4 * NBUF]
        wid = lax.axis_index("s") * NC + lax.axis_index("c")
        e0 = wid * EW
        pltpu.sync_copy(cidx_hbm.at[pl.ds(e_off + e0, EW)], cidx_all)
        pltpu.sync_copy(nidx_hbm.at[pl.ds(e_off + e0, EW)], nidx_all)

        def g_start(i, b):
            pltpu.async_copy(
                x_hbm.at[cidx_all.at[pl.ds(i * CH, CH)]], crow[b], gc[b])
            pltpu.async_copy(
                x_hbm.at[nidx_all.at[pl.ds(i * CH, CH)]], nrow[b], gn[b])

        def g_wait(b):
            pltpu.make_async_copy(x_hbm.at[pl.ds(0, CH), :], crow[b], gc[b]).wait()
            pltpu.make_async_copy(x_hbm.at[pl.ds(0, CH), :], nrow[b], gn[b]).wait()

        def w_start(i, b):
            pltpu.async_copy(crow[b], cout_hbm.at[pl.ds(e0 + i * CH, CH), :], wc[b])
            pltpu.async_copy(nrow[b], nout_hbm.at[pl.ds(e0 + i * CH, CH), :], wn[b])

        def w_wait(b):
            pltpu.make_async_copy(crow[b], cout_hbm.at[pl.ds(0, CH), :], wc[b]).wait()
            pltpu.make_async_copy(nrow[b], nout_hbm.at[pl.ds(0, CH), :], wn[b]).wait()

        for b in range(NBUF):
            g_start(b, b)

        def outer(j, carry):
            for b in range(NBUF):
                i = j * NBUF + b

                @pl.when(i < n_chunks)
                def _():
                    g_wait(b)
                    w_start(i, b)
                    w_wait(b)

                    @pl.when(i + NBUF < n_chunks)
                    def _():
                        g_start(i + NBUF, b)
            return carry

        lax.fori_loop(0, n_outer, outer, 0)

    return gather_k


SBUF = 4


@functools.lru_cache(maxsize=None)
def _make_scatter(N, D, EP, e_off=0):
    NBUF = SBUF
    NW = NC * NS
    P = 1
    EW = EP // NW
    n_chunks = EW // CH
    NR = ((N // NS + 7) // 8) * 8
    NR_LAST = N - NR * (NS - 1)
    assert NR_LAST > 0 and NR_LAST % 8 == 0

    n_outer = (n_chunks + NBUF - 1) // NBUF
    assert n_chunks >= NBUF
    scratch = (
        [pltpu.VMEM((CH,), jnp.int32)] * NBUF
        + [pltpu.VMEM((CH, D), jnp.float32)] * NBUF
        + [pltpu.VMEM_SHARED((N, D), jnp.float32)]
        + [pltpu.SemaphoreType.DMA] * (3 * NBUF)
    )

    @functools.partial(
        pl.kernel,
        out_type=jax.ShapeDtypeStruct((NC, N, D), jnp.float32),
        mesh=_sc_mesh(),
        scratch_types=scratch)
    def scatter_k(cidx_hbm, zeros_hbm, *rest):
        msgs_hbm = rest[:P]
        agg_hbm = rest[P]
        scr = rest[P + 1:]
        cidx_v = scr[0:NBUF]
        msg_v = scr[NBUF:2 * NBUF]
        agg_sh = scr[2 * NBUF]
        ic = scr[2 * NBUF + 1:2 * NBUF + 1 + NBUF]
        im = scr[2 * NBUF + 1 + NBUF:2 * NBUF + 1 + 2 * NBUF]
        ss = scr[2 * NBUF + 1 + 2 * NBUF:2 * NBUF + 1 + 3 * NBUF]
        c = lax.axis_index("c")
        s = lax.axis_index("s")
        wid = s * NC + c
        e0 = wid * EW
        @pl.when(s < NS - 1)
        def _():
            pltpu.sync_copy(zeros_hbm.at[pl.ds(s * NR, NR), :],
                            agg_sh.at[pl.ds(s * NR, NR), :])

        @pl.when(s == NS - 1)
        def _():
            pltpu.sync_copy(zeros_hbm.at[pl.ds((NS - 1) * NR, NR_LAST), :],
                            agg_sh.at[pl.ds((NS - 1) * NR, NR_LAST), :])

        plsc.subcore_barrier()

        def l_wait(b):
            pltpu.make_async_copy(cidx_hbm.at[pl.ds(0, CH)], cidx_v[b], ic[b]).wait()
            pltpu.make_async_copy(
                msgs_hbm[0].at[pl.ds(0, CH), :], msg_v[b], im[b]).wait()

        def s_start(b):
            pltpu.async_copy(msg_v[b], agg_sh.at[cidx_v[b]], ss[b], add=True)

        def s_wait(b):
            pltpu.make_async_copy(
                msgs_hbm[0].at[pl.ds(0, CH), :], msg_v[b], ss[b]).wait()

        for p in range(P):
            msg_hbm = msgs_hbm[p]

            def l_start(i, b, p=p, msg_hbm=msg_hbm):
                pltpu.async_copy(
                    cidx_hbm.at[pl.ds(e_off + e0 + i * CH, CH)], cidx_v[b], ic[b])
                pltpu.async_copy(
                    msg_hbm.at[pl.ds(e0 + i * CH, CH), :], msg_v[b], im[b])

            for b in range(NBUF):
                l_start(b, b)

            def outer(j, carry, l_start=l_start):
                for b in range(NBUF):
                    i = j * NBUF + b

                    @pl.when(i < n_chunks)
                    def _():
                        l_wait(b)
                        s_start(b)
                        s_wait(b)

                        @pl.when(i + NBUF < n_chunks)
                        def _():
                            l_start(i + NBUF, b)
                return carry

            lax.fori_loop(0, n_outer, outer, 0)
        plsc.subcore_barrier()

        @pl.when(s < NS - 1)
        def _():
            pltpu.sync_copy(agg_sh.at[pl.ds(s * NR, NR), :],
                            agg_hbm.at[c, pl.ds(s * NR, NR), :])

        @pl.when(s == NS - 1)
        def _():
            pltpu.sync_copy(agg_sh.at[pl.ds((NS - 1) * NR, NR_LAST), :],
                            agg_hbm.at[c, pl.ds((NS - 1) * NR, NR_LAST), :])

    return scatter_k


def kernel(node_embeddings, edge_embeddings, edge_index_list, ln_g, ln_b,
           phi_w1, phi_b1, phi_w2, phi_b2, theta_w, theta_b):
    N, D = node_embeddings.shape
    E = edge_index_list.shape[1]
    H = phi_w1.shape[0]
    assert E % (NC * NS * CH) == 0 and N % NS == 0

    BN = 1000
    assert N % BN == 0
    x = pl.pallas_call(
        _ln_body,
        grid=(N // BN,),
        in_specs=[
            pl.BlockSpec((BN, D), lambda i: (i, 0)),
            pl.BlockSpec((1, D), lambda i: (0, 0)),
            pl.BlockSpec((1, D), lambda i: (0, 0)),
        ],
        out_specs=pl.BlockSpec((BN, D), lambda i: (i, 0)),
        out_shape=jax.ShapeDtypeStruct((N, D), jnp.float32),
    )(node_embeddings, ln_g.reshape(1, D), ln_b.reshape(1, D))

    P = 5
    EP = E // P
    cidx = edge_index_list[0]
    nidx = edge_index_list[1]
    BE = 2560
    assert EP % BE == 0 and E % (P * NC * NS * CH) == 0
    w1t = phi_w1.T.astype(jnp.bfloat16)
    w2t = phi_w2.T.astype(jnp.bfloat16)

    def mlp_fn(p):
        off = p * (EP // BE)
        return pl.pallas_call(
            _mlp_body,
            grid=(EP // BE,),
            in_specs=[
                pl.BlockSpec((BE, D), lambda i: (off + i, 0)),
                pl.BlockSpec((BE, D), lambda i: (i, 0)),
                pl.BlockSpec((BE, D), lambda i: (i, 0)),
                pl.BlockSpec((1, D), lambda i: (0, 0)),
                pl.BlockSpec((1, D), lambda i: (0, 0)),
                pl.BlockSpec((D, H), lambda i: (0, 0)),
                pl.BlockSpec((1, H), lambda i: (0, 0)),
                pl.BlockSpec((H, D), lambda i: (0, 0)),
                pl.BlockSpec((1, D), lambda i: (0, 0)),
            ],
            out_specs=pl.BlockSpec((BE, D), lambda i: (i, 0)),
            out_shape=jax.ShapeDtypeStruct((EP, D), jnp.float32),
            compiler_params=pltpu.CompilerParams(
                dimension_semantics=("arbitrary",)),
        )

    zeros = jnp.zeros((N, D), jnp.float32)
    msgs = []
    aggps = []
    tok = x[0, 0]

    def launch_scatter(q, tok):
        cidx_q, _ = lax.optimization_barrier((cidx, tok))
        aggps.append(_make_scatter(N, D, EP, q * EP)(cidx_q, zeros, msgs[q]))
        return aggps[-1][0, 0, 0]

    tok = node_embeddings[0, 0]
    for p in range(P):
        cidx_p, _ = lax.optimization_barrier((cidx, tok))
        crows, nrows = _make_gather(N, D, EP, p * EP)(
            node_embeddings, cidx_p, nidx)
        tok = crows[0, 0]
        msgs.append(mlp_fn(p)(
            edge_embeddings, crows, nrows, ln_g.reshape(1, D),
            ln_b.reshape(1, D), w1t, phi_b1.reshape(1, H),
            w2t, phi_b2.reshape(1, D)))
        if p >= 1:
            tok = launch_scatter(p - 1, tok)
    tok = launch_scatter(P - 1, tok)

    wt = theta_w.T
    out = pl.pallas_call(
        _out_body,
        grid=(N // BN,),
        in_specs=(
            [pl.BlockSpec((BN, D), lambda i: (i, 0))]
            + [pl.BlockSpec((NC, BN, D), lambda i: (0, i, 0))] * P
            + [pl.BlockSpec((D, D), lambda i: (0, 0)),
               pl.BlockSpec((1, D), lambda i: (0, 0))]
        ),
        out_specs=pl.BlockSpec((BN, D), lambda i: (i, 0)),
        out_shape=jax.ShapeDtypeStruct((N, D), jnp.float32),
    )(x, *aggps, wt, theta_b.reshape(1, D))
    return out
